# baseline (speedup 1.0000x reference)
import jax
import jax.numpy as jnp
from jax.experimental import pallas as pl
from jax.experimental.pallas import tpu as pltpu


def _pos_kernel(col_ref, row_ref, out_ref, buf0, buf1, sems):
    batch = out_ref.shape[0]
    h = out_ref.shape[2]
    w = out_ref.shape[3]
    cblk = buf0.shape[0]
    embed_dim = col_ref.shape[1]
    n_stages = out_ref.shape[1] // cblk
    nx = embed_dim // cblk
    bufs = [buf0, buf1]

    xt = col_ref[:w, :].T
    yt = row_ref[:h, :].T

    def copies(k):
        buf = bufs[k % 2]
        return [
            pltpu.make_async_copy(
                buf,
                out_ref.at[b, pl.ds(k * cblk, cblk), :, :],
                sems.at[k % 2, b],
            )
            for b in range(batch)
        ]

    for k in range(n_stages):
        if k >= 2:
            for c in copies(k - 2):
                c.wait()
        buf = bufs[k % 2]
        if k < nx:
            blk = xt[k * cblk : (k + 1) * cblk, :]
            buf[...] = jnp.broadcast_to(blk[:, None, :], (cblk, h, w))
        else:
            blk = yt[(k - nx) * cblk : (k - nx + 1) * cblk, :]
            buf[...] = jnp.broadcast_to(blk[:, :, None], (cblk, h, w))
        for c in copies(k):
            c.start()

    for k in range(max(n_stages - 2, 0), n_stages):
        for c in copies(k):
            c.wait()


def kernel(pixel_values, row_weight, col_weight):
    batch = pixel_values.shape[0]
    height, width = pixel_values.shape[-2], pixel_values.shape[-1]
    embed_dim = row_weight.shape[1]
    cblk = 128

    out = pl.pallas_call(
        _pos_kernel,
        in_specs=[
            pl.BlockSpec(memory_space=pltpu.MemorySpace.VMEM),
            pl.BlockSpec(memory_space=pltpu.MemorySpace.VMEM),
        ],
        out_specs=pl.BlockSpec(memory_space=pltpu.MemorySpace.HBM),
        out_shape=jax.ShapeDtypeStruct(
            (batch, 2 * embed_dim, height, width), jnp.float32
        ),
        scratch_shapes=[
            pltpu.VMEM((cblk, height, width), jnp.float32),
            pltpu.VMEM((cblk, height, width), jnp.float32),
            pltpu.SemaphoreType.DMA((2, batch)),
        ],
    )(col_weight, row_weight)
    return out


# speedup vs baseline: 1.2743x; 1.2743x over previous
import jax
import jax.numpy as jnp
from jax.experimental import pallas as pl
from jax.experimental.pallas import tpu as pltpu


def _pos_kernel(col_ref, row_ref, out_ref, buf0, buf1, sems):
    batch = out_ref.shape[0]
    hw = out_ref.shape[2]
    cblk = buf0.shape[0]
    embed_dim = col_ref.shape[1]
    n_stages = out_ref.shape[1] // cblk
    nx = embed_dim // cblk
    w = col_ref.shape[0]
    h = row_ref.shape[0]
    bufs = [buf0, buf1]

    xt = col_ref[...].T
    yt = row_ref[...].T

    def copies(k):
        buf = bufs[k % 2]
        return [
            pltpu.make_async_copy(
                buf,
                out_ref.at[b, pl.ds(k * cblk, cblk), :],
                sems.at[k % 2, b],
            )
            for b in range(batch)
        ]

    for k in range(n_stages):
        if k >= 2:
            for c in copies(k - 2):
                c.wait()
        buf = bufs[k % 2]
        if k < nx:
            blk = xt[k * cblk : (k + 1) * cblk, :]
            buf[...] = jnp.broadcast_to(
                blk[:, None, :], (cblk, h, w)
            ).reshape(cblk, hw)
        else:
            blk = yt[(k - nx) * cblk : (k - nx + 1) * cblk, :]
            buf[...] = jnp.broadcast_to(
                blk[:, :, None], (cblk, h, w)
            ).reshape(cblk, hw)
        for c in copies(k):
            c.start()

    for k in range(max(n_stages - 2, 0), n_stages):
        for c in copies(k):
            c.wait()


def kernel(pixel_values, row_weight, col_weight):
    batch = pixel_values.shape[0]
    height, width = pixel_values.shape[-2], pixel_values.shape[-1]
    embed_dim = row_weight.shape[1]
    cblk = 128

    out = pl.pallas_call(
        _pos_kernel,
        in_specs=[
            pl.BlockSpec(memory_space=pltpu.MemorySpace.VMEM),
            pl.BlockSpec(memory_space=pltpu.MemorySpace.VMEM),
        ],
        out_specs=pl.BlockSpec(memory_space=pltpu.MemorySpace.HBM),
        out_shape=jax.ShapeDtypeStruct(
            (batch, 2 * embed_dim, height * width), jnp.float32
        ),
        scratch_shapes=[
            pltpu.VMEM((cblk, height * width), jnp.float32),
            pltpu.VMEM((cblk, height * width), jnp.float32),
            pltpu.SemaphoreType.DMA((2, batch)),
        ],
    )(col_weight[:width, :], row_weight[:height, :])
    return out.reshape(batch, 2 * embed_dim, height, width)
